# baseline (speedup 1.0000x reference)
import functools

import jax
import jax.numpy as jnp
from jax import lax
from jax.experimental import pallas as pl
from jax.experimental.pallas import tpu as pltpu
from jax.experimental.pallas import tpu_sc as plsc

B = 2048
D = 2048
N = 16384
K = 64

NC = 2
NS = 16
L = 16
NW = NC * NS
RPW = B // NW

NGROUP = 64
GCH = 16
NCHUNK = K // GCH
SV = 16
NSTRIPE = D // (SV * L)

_BIG = 1 << 20


def _enc_body(x_ref, w_ref, bdec_ref, benc_ref, o_ref):
    xm = x_ref[...] - bdec_ref[...]
    acc = lax.dot_general(xm, w_ref[...], (((1,), (1,)), ((), ())),
                          preferred_element_type=jnp.float32)
    o_ref[...] = jnp.maximum(acc + benc_ref[...], 0.0)


def _encode(x, W_enc, b_dec2, b_enc2):
    NBLK = 512
    return pl.pallas_call(
        _enc_body,
        grid=(N // NBLK,),
        in_specs=[
            pl.BlockSpec((B, D), lambda i: (0, 0)),
            pl.BlockSpec((NBLK, D), lambda i: (i, 0)),
            pl.BlockSpec((1, D), lambda i: (0, 0)),
            pl.BlockSpec((1, NBLK), lambda i: (0, i)),
        ],
        out_specs=pl.BlockSpec((B, NBLK), lambda i: (0, i)),
        out_shape=jax.ShapeDtypeStruct((B, N), jnp.float32),
    )(x, W_enc, b_dec2, b_enc2)


def _sc_body(pre_hbm, wdec_hbm, vals_hbm, cols_hbm, out_hbm,
             rb0, rb1, l1_v, vals0, vals1, cols0, cols1, gA, gB, acc_v,
             acc2_v, sem_r0, sem_r1, sem_gA, sem_gB):
    wid = lax.axis_index("s") * NC + lax.axis_index("c")
    row0 = wid * RPW
    iota = lax.iota(jnp.int32, L)
    lane0 = iota == 0

    def _sets(ref, idx_scalars, val):
        idxs = [jnp.full((L,), i, jnp.int32) for i in idx_scalars]
        plsc.store_scatter(ref, idxs, jnp.full((L,), val), mask=lane0)

    def _extract(rb, vals_r, cols_r):
        def build(j, l2):
            base = j * 256
            acc = rb[pl.ds(base, L)]
            for i in range(1, 16):
                acc = jnp.maximum(acc, rb[pl.ds(base + i * L, L)])
            l1_v[pl.ds(j * L, L)] = acc
            return jnp.maximum(l2, acc)

        l2 = lax.fori_loop(0, NGROUP, build,
                           jnp.full((L,), -1.0, jnp.float32))

        def ext(t, l2):
            m = jnp.max(l2)
            lane = jnp.min(jnp.where(l2 == m, iota, L))
            best = jnp.zeros((L,), jnp.int32)
            vs = []
            for q in range(NGROUP // L):
                idx = (iota + q * L) * L + lane
                v = plsc.load_gather(l1_v, [idx])
                vs.append(v)
                best = jnp.maximum(
                    best, jnp.where(v == m, _BIG - (iota + q * L), 0))
            j = _BIG - jnp.max(best)
            cidx = j * 256 + iota * L + lane
            cand = plsc.load_gather(rb, [cidx])
            i_ = _BIG - jnp.max(jnp.where(cand == m, _BIG - iota, 0))
            col = j * 256 + i_ * L + lane
            _sets(vals_r, [t], m)
            _sets(cols_r, [t // GCH, t % GCH], col)
            _sets(rb, [col], jnp.float32(-1.0))
            newcand = jnp.where(iota == i_, jnp.float32(-1.0), cand)
            g1 = jnp.max(newcand)
            _sets(l1_v, [j * L + lane], g1)
            b2 = jnp.full((L,), -1.0, jnp.float32)
            for q in range(NGROUP // L):
                b2 = jnp.maximum(
                    b2, jnp.where(iota + q * L == j, g1, vs[q]))
            return jnp.where(iota == lane, jnp.max(b2), l2)

        lax.fori_loop(0, K, ext, l2)

    def _fire(cols_r, c, gb, sem):
        pltpu.async_copy(wdec_hbm.at[cols_r.at[c]], gb, sem)

    def _accum(gb, vals_r, c):
        def sbody(s, _):
            base = s * SV * L
            if c == 0:
                regs = tuple(jnp.zeros((L,), jnp.float32) for _ in range(SV))
            else:
                regs = tuple(acc_v[pl.ds(base + i * L, L)] for i in range(SV))

            def kb(k, regs):
                a = plsc.load_gather(
                    vals_r, [jnp.full((L,), c * GCH + k, jnp.int32)])
                new = []
                for u in range(SV // 2):
                    w = gb[k, pl.ds(base // 2 + u * L, L)]
                    wa, wb = plsc.unpack(
                        plsc.bitcast(w, jnp.bfloat16),
                        format=plsc.PackFormat.INTERLEAVED)
                    new.append(regs[2 * u] + a * wa)
                    new.append(regs[2 * u + 1] + a * wb)
                return tuple(new)

            regs = lax.fori_loop(0, GCH, kb, regs)
            for i in range(SV):
                acc_v[pl.ds(base + i * L, L)] = regs[i]
            return 0

        lax.fori_loop(0, NSTRIPE, sbody, 0)

    def _decode(vals_r, cols_r, row):
        for c in range(NCHUNK):
            gb, sem = (gA, sem_gA) if c % 2 == 0 else (gB, sem_gB)
            pltpu.make_async_copy(wdec_hbm.at[cols_r.at[c]], gb, sem).wait()
            _accum(gb, vals_r, c)
            if c + 2 < NCHUNK:
                _fire(cols_r, c + 2, gb, sem)
        def deint(b32, _):
            e = acc_v[pl.ds(b32 * 32, L)]
            o = acc_v[pl.ds(b32 * 32 + L, L)]
            plsc.store_scatter(acc2_v, [b32 * 32 + 2 * iota], e)
            plsc.store_scatter(acc2_v, [b32 * 32 + 2 * iota + 1], o)
            return 0

        lax.fori_loop(0, D // 32, deint, 0)
        pltpu.sync_copy(acc2_v, out_hbm.at[row])
        pltpu.sync_copy(vals_r, vals_hbm.at[row])
        pltpu.sync_copy(cols_r, cols_hbm.at[row])

    pltpu.sync_copy(pre_hbm.at[row0], rb0)
    pltpu.async_copy(pre_hbm.at[row0 + 1], rb1, sem_r1)
    _extract(rb0, vals0, cols0)

    def pair_body(i, _):
        r = row0 + 2 * i

        @pl.when(2 * i + 2 < RPW)
        def _():
            pltpu.async_copy(pre_hbm.at[r + 2], rb0, sem_r0)

        _fire(cols0, 0, gA, sem_gA)
        _fire(cols0, 1, gB, sem_gB)
        pltpu.make_async_copy(pre_hbm.at[r + 1], rb1, sem_r1).wait()
        _extract(rb1, vals1, cols1)
        _decode(vals0, cols0, r)

        @pl.when(2 * i + 3 < RPW)
        def _():
            pltpu.async_copy(pre_hbm.at[r + 3], rb1, sem_r1)

        _fire(cols1, 0, gA, sem_gA)
        _fire(cols1, 1, gB, sem_gB)

        @pl.when(2 * i + 2 < RPW)
        def _():
            pltpu.make_async_copy(pre_hbm.at[r + 2], rb0, sem_r0).wait()
            _extract(rb0, vals0, cols0)

        _decode(vals1, cols1, r + 1)
        return 0

    lax.fori_loop(0, RPW // 2, pair_body, 0)


def _sc_topk_decode(pre, W_dec):
    mesh = plsc.VectorSubcoreMesh(core_axis_name="c", subcore_axis_name="s")
    fn = functools.partial(
        pl.kernel,
        out_type=[
            jax.ShapeDtypeStruct((B, K), jnp.float32),
            jax.ShapeDtypeStruct((B, NCHUNK, GCH), jnp.int32),
            jax.ShapeDtypeStruct((B, D), jnp.float32),
        ],
        mesh=mesh,
        scratch_types=[
            pltpu.VMEM((N,), jnp.float32),
            pltpu.VMEM((N,), jnp.float32),
            pltpu.VMEM((NGROUP * L,), jnp.float32),
            pltpu.VMEM((K,), jnp.float32),
            pltpu.VMEM((K,), jnp.float32),
            pltpu.VMEM((NCHUNK, GCH), jnp.int32),
            pltpu.VMEM((NCHUNK, GCH), jnp.int32),
            pltpu.VMEM((GCH, D // 2), jnp.float32),
            pltpu.VMEM((GCH, D // 2), jnp.float32),
            pltpu.VMEM((D,), jnp.float32),
            pltpu.VMEM((D,), jnp.float32),
            pltpu.SemaphoreType.DMA,
            pltpu.SemaphoreType.DMA,
            pltpu.SemaphoreType.DMA,
            pltpu.SemaphoreType.DMA,
        ],
        compiler_params=pltpu.CompilerParams(needs_layout_passes=False),
    )(_sc_body)
    return fn(pre, W_dec)


def _stats_body(x_ref, tv_ref, colsum_ref, s_ref):
    i = pl.program_id(0)

    @pl.when(i == 0)
    def _():
        colsum_ref[...] = jnp.zeros_like(colsum_ref)
        s_ref[0] = 0.0

    x = x_ref[...]
    colsum_ref[...] += jnp.sum(x, axis=0, keepdims=True)
    s_ref[0] += jnp.sum(x * x)
    nb = pl.num_programs(0)

    @pl.when(i == nb - 1)
    def _():
        cs = colsum_ref[...]
        tv = s_ref[0] - jnp.sum(cs * cs) / B
        tv_ref[...] = jnp.full((1, 1), tv, jnp.float32)


def _stats(x):
    RB = 256
    return pl.pallas_call(
        _stats_body,
        grid=(B // RB,),
        in_specs=[pl.BlockSpec((RB, D), lambda i: (i, 0))],
        out_specs=pl.BlockSpec((1, 1), lambda i: (0, 0)),
        out_shape=jax.ShapeDtypeStruct((1, 1), jnp.float32),
        scratch_shapes=[
            pltpu.VMEM((1, D), jnp.float32),
            pltpu.SMEM((1,), jnp.float32),
        ],
    )(x)


def _fin_body(x_ref, p_ref, bdec_ref, tv_ref, sae_ref, fvu_ref, s_ref):
    i = pl.program_id(0)

    @pl.when(i == 0)
    def _():
        s_ref[0] = 0.0

    x = x_ref[...]
    sae = p_ref[...] + bdec_ref[...]
    sae_ref[...] = sae
    e = x - sae
    s_ref[0] += jnp.sum(e * e)
    nb = pl.num_programs(0)

    @pl.when(i == nb - 1)
    def _():
        fvu_ref[...] = jnp.full((1, 1), s_ref[0], jnp.float32) / tv_ref[...]


def _finalize(x, partial, b_dec2, tv):
    RB = 256
    return pl.pallas_call(
        _fin_body,
        grid=(B // RB,),
        in_specs=[
            pl.BlockSpec((RB, D), lambda i: (i, 0)),
            pl.BlockSpec((RB, D), lambda i: (i, 0)),
            pl.BlockSpec((1, D), lambda i: (0, 0)),
            pl.BlockSpec((1, 1), lambda i: (0, 0)),
        ],
        out_specs=[
            pl.BlockSpec((RB, D), lambda i: (i, 0)),
            pl.BlockSpec((1, 1), lambda i: (0, 0)),
        ],
        out_shape=[
            jax.ShapeDtypeStruct((B, D), jnp.float32),
            jax.ShapeDtypeStruct((1, 1), jnp.float32),
        ],
        scratch_shapes=[
            pltpu.SMEM((1,), jnp.float32),
        ],
    )(x, partial, b_dec2, tv)


def kernel(x, W_enc, b_enc, W_dec, b_dec):
    b_dec2 = b_dec.reshape(1, D)
    b_enc2 = b_enc.reshape(1, N)
    wdec_bf = W_dec.reshape(N, D // 2, 2).astype(jnp.bfloat16)
    wdec_pk = lax.bitcast_convert_type(wdec_bf, jnp.float32)
    pre = _encode(x, W_enc, b_dec2, b_enc2)
    tv = _stats(x)
    top_acts, cols3, partial = _sc_topk_decode(pre, wdec_pk)
    top_indices = cols3.reshape(B, K)
    sae_out, fvu2 = _finalize(x, partial, b_dec2, tv)
    fvu = fvu2[0, 0]
    zero = jnp.zeros((), x.dtype)
    return (sae_out, top_acts, top_indices, fvu, zero, zero)


# speedup vs baseline: 1.4286x; 1.4286x over previous
import functools

import jax
import jax.numpy as jnp
from jax import lax
from jax.experimental import pallas as pl
from jax.experimental.pallas import tpu as pltpu
from jax.experimental.pallas import tpu_sc as plsc

B = 2048
D = 2048
N = 16384
K = 64

NC = 2
NS = 16
L = 16
NW = NC * NS
RPW = B // NW

NGROUP = 64
GCH = 8
NCHUNK = K // GCH
SV = 16
NSTRIPE = D // (SV * L)
NPAIR = RPW // 2

_BIG = 1 << 20


def _enc_body(x_ref, w_ref, bdec_ref, benc_ref, o_ref):
    xm = x_ref[...] - bdec_ref[...]
    acc = lax.dot_general(xm, w_ref[...], (((1,), (1,)), ((), ())),
                          preferred_element_type=jnp.float32)
    o_ref[...] = jnp.maximum(acc + benc_ref[...], 0.0)


def _encode(x, W_enc, b_dec2, b_enc2):
    NBLK = 512
    return pl.pallas_call(
        _enc_body,
        grid=(N // NBLK,),
        in_specs=[
            pl.BlockSpec((B, D), lambda i: (0, 0)),
            pl.BlockSpec((NBLK, D), lambda i: (i, 0)),
            pl.BlockSpec((1, D), lambda i: (0, 0)),
            pl.BlockSpec((1, NBLK), lambda i: (0, i)),
        ],
        out_specs=pl.BlockSpec((B, NBLK), lambda i: (0, i)),
        out_shape=jax.ShapeDtypeStruct((B, N), jnp.float32),
    )(x, W_enc, b_dec2, b_enc2)


def _sc_body(pre_hbm, wdec_hbm, vals_hbm, cols_hbm, out_hbm,
             rb0, rb1, rb2, rb3, l1a_v, l1b_v,
             vals0, vals1, vals2, vals3, cols0, cols1, cols2, cols3,
             gA, gB, acc_v,
             sem_r0, sem_r1, sem_r2, sem_r3, sem_gA, sem_gB):
    wid = lax.axis_index("s") * NC + lax.axis_index("c")
    row0 = wid * RPW
    iota = lax.iota(jnp.int32, L)
    lane0 = iota == 0
    rb = (rb0, rb1, rb2, rb3)
    vals = (vals0, vals1, vals2, vals3)
    cols = (cols0, cols1, cols2, cols3)
    sem_r = (sem_r0, sem_r1, sem_r2, sem_r3)

    def _sets(ref, idx_scalars, val):
        idxs = [jnp.full((L,), i, jnp.int32) for i in idx_scalars]
        plsc.store_scatter(ref, idxs, jnp.full((L,), val), mask=lane0)

    def _ext_one(rbx, l1, vals_r, cols_r, t, l2):
        m = jnp.max(l2)
        lane = jnp.min(jnp.where(l2 == m, iota, L))
        best = jnp.zeros((L,), jnp.int32)
        vs = []
        for q in range(NGROUP // L):
            idx = (iota + q * L) * L + lane
            v = plsc.load_gather(l1, [idx])
            vs.append(v)
            best = jnp.maximum(
                best, jnp.where(v == m, _BIG - (iota + q * L), 0))
        j = _BIG - jnp.max(best)
        cidx = j * 256 + iota * L + lane
        cand = plsc.load_gather(rbx, [cidx])
        i_ = _BIG - jnp.max(jnp.where(cand == m, _BIG - iota, 0))
        col = j * 256 + i_ * L + lane
        _sets(vals_r, [t], m)
        _sets(cols_r, [t // GCH, t % GCH], col)
        _sets(rbx, [col], jnp.float32(-1.0))
        newcand = jnp.where(iota == i_, jnp.float32(-1.0), cand)
        g1 = jnp.max(newcand)
        _sets(l1, [j * L + lane], g1)
        b2 = jnp.full((L,), -1.0, jnp.float32)
        for q in range(NGROUP // L):
            b2 = jnp.maximum(b2, jnp.where(iota + q * L == j, g1, vs[q]))
        return jnp.where(iota == lane, jnp.max(b2), l2)

    def _extract2(rbA, rbB, valsA, colsA, valsB, colsB):
        def build(j, carry):
            l2a, l2b = carry
            base = j * 256
            accA = rbA[pl.ds(base, L)]
            accB = rbB[pl.ds(base, L)]
            for i in range(1, 16):
                accA = jnp.maximum(accA, rbA[pl.ds(base + i * L, L)])
                accB = jnp.maximum(accB, rbB[pl.ds(base + i * L, L)])
            l1a_v[pl.ds(j * L, L)] = accA
            l1b_v[pl.ds(j * L, L)] = accB
            return (jnp.maximum(l2a, accA), jnp.maximum(l2b, accB))

        init = (jnp.full((L,), -1.0, jnp.float32),
                jnp.full((L,), -1.0, jnp.float32))
        l2a, l2b = lax.fori_loop(0, NGROUP, build, init)

        def ext(t, carry):
            l2a, l2b = carry
            return (_ext_one(rbA, l1a_v, valsA, colsA, t, l2a),
                    _ext_one(rbB, l1b_v, valsB, colsB, t, l2b))

        lax.fori_loop(0, K, ext, (l2a, l2b))

    def _fire(cols_r, c, gb, sem):
        pltpu.async_copy(wdec_hbm.at[cols_r.at[c]], gb, sem)

    def _accum(gb, vals_r, c):
        def sbody(st, _):
            base = st * SV * L
            if c == 0:
                regs = tuple(jnp.zeros((L,), jnp.float32) for _ in range(SV))
            else:
                regs = tuple(acc_v[pl.ds(base + i * L, L)] for i in range(SV))

            def kb(k, regs):
                a = plsc.load_gather(
                    vals_r, [jnp.full((L,), c * GCH + k, jnp.int32)])
                return tuple(
                    regs[i] + a * gb[k, pl.ds(base + i * L, L)]
                    for i in range(SV))

            regs = lax.fori_loop(0, GCH, kb, regs)
            for i in range(SV):
                acc_v[pl.ds(base + i * L, L)] = regs[i]
            return 0

        lax.fori_loop(0, NSTRIPE, sbody, 0)

    def _decode(vals_r, cols_r, row):
        for c in range(NCHUNK):
            gb, sem = (gA, sem_gA) if c % 2 == 0 else (gB, sem_gB)
            pltpu.make_async_copy(wdec_hbm.at[cols_r.at[c]], gb, sem).wait()
            _accum(gb, vals_r, c)
            if c + 2 < NCHUNK:
                _fire(cols_r, c + 2, gb, sem)
        pltpu.sync_copy(acc_v, out_hbm.at[row])
        pltpu.sync_copy(vals_r, vals_hbm.at[row])
        pltpu.sync_copy(cols_r, cols_hbm.at[row])

    pltpu.sync_copy(pre_hbm.at[row0], rb0)
    pltpu.sync_copy(pre_hbm.at[row0 + 1], rb1)
    pltpu.async_copy(pre_hbm.at[row0 + 2], rb2, sem_r2)
    pltpu.async_copy(pre_hbm.at[row0 + 3], rb3, sem_r3)
    _extract2(rb0, rb1, vals0, cols0, vals1, cols1)

    def pair_step(p, sg):
        r = row0 + 2 * p
        a = 2 * sg
        na = 2 - 2 * sg

        @pl.when(p + 2 < NPAIR)
        def _():
            pltpu.async_copy(pre_hbm.at[r + 4], rb[a], sem_r[a])
            pltpu.async_copy(pre_hbm.at[r + 5], rb[a + 1], sem_r[a + 1])

        _fire(cols[a], 0, gA, sem_gA)
        _fire(cols[a], 1, gB, sem_gB)

        @pl.when(p + 1 < NPAIR)
        def _():
            pltpu.make_async_copy(pre_hbm.at[r + 2], rb[na], sem_r[na]).wait()
            pltpu.make_async_copy(
                pre_hbm.at[r + 3], rb[na + 1], sem_r[na + 1]).wait()
            _extract2(rb[na], rb[na + 1], vals[na], cols[na],
                      vals[na + 1], cols[na + 1])

        _decode(vals[a], cols[a], r)
        _fire(cols[a + 1], 0, gA, sem_gA)
        _fire(cols[a + 1], 1, gB, sem_gB)
        _decode(vals[a + 1], cols[a + 1], r + 1)

    def body(i2, _):
        pair_step(2 * i2, 0)
        pair_step(2 * i2 + 1, 1)
        return 0

    lax.fori_loop(0, NPAIR // 2, body, 0)


def _sc_topk_decode(pre, W_dec):
    mesh = plsc.VectorSubcoreMesh(core_axis_name="c", subcore_axis_name="s")
    fn = functools.partial(
        pl.kernel,
        out_type=[
            jax.ShapeDtypeStruct((B, K), jnp.float32),
            jax.ShapeDtypeStruct((B, NCHUNK, GCH), jnp.int32),
            jax.ShapeDtypeStruct((B, D), jnp.float32),
        ],
        mesh=mesh,
        scratch_types=[
            pltpu.VMEM((N,), jnp.float32),
            pltpu.VMEM((N,), jnp.float32),
            pltpu.VMEM((N,), jnp.float32),
            pltpu.VMEM((N,), jnp.float32),
            pltpu.VMEM((NGROUP * L,), jnp.float32),
            pltpu.VMEM((NGROUP * L,), jnp.float32),
            pltpu.VMEM((K,), jnp.float32),
            pltpu.VMEM((K,), jnp.float32),
            pltpu.VMEM((K,), jnp.float32),
            pltpu.VMEM((K,), jnp.float32),
            pltpu.VMEM((NCHUNK, GCH), jnp.int32),
            pltpu.VMEM((NCHUNK, GCH), jnp.int32),
            pltpu.VMEM((NCHUNK, GCH), jnp.int32),
            pltpu.VMEM((NCHUNK, GCH), jnp.int32),
            pltpu.VMEM((GCH, D), jnp.float32),
            pltpu.VMEM((GCH, D), jnp.float32),
            pltpu.VMEM((D,), jnp.float32),
            pltpu.SemaphoreType.DMA,
            pltpu.SemaphoreType.DMA,
            pltpu.SemaphoreType.DMA,
            pltpu.SemaphoreType.DMA,
            pltpu.SemaphoreType.DMA,
            pltpu.SemaphoreType.DMA,
        ],
        compiler_params=pltpu.CompilerParams(needs_layout_passes=False),
    )(_sc_body)
    return fn(pre, W_dec)


def _stats_body(x_ref, tv_ref, colsum_ref, s_ref):
    i = pl.program_id(0)

    @pl.when(i == 0)
    def _():
        colsum_ref[...] = jnp.zeros_like(colsum_ref)
        s_ref[0] = 0.0

    x = x_ref[...]
    colsum_ref[...] += jnp.sum(x, axis=0, keepdims=True)
    s_ref[0] += jnp.sum(x * x)
    nb = pl.num_programs(0)

    @pl.when(i == nb - 1)
    def _():
        cs = colsum_ref[...]
        tv = s_ref[0] - jnp.sum(cs * cs) / B
        tv_ref[...] = jnp.full((1, 1), tv, jnp.float32)


def _stats(x):
    RB = 256
    return pl.pallas_call(
        _stats_body,
        grid=(B // RB,),
        in_specs=[pl.BlockSpec((RB, D), lambda i: (i, 0))],
        out_specs=pl.BlockSpec((1, 1), lambda i: (0, 0)),
        out_shape=jax.ShapeDtypeStruct((1, 1), jnp.float32),
        scratch_shapes=[
            pltpu.VMEM((1, D), jnp.float32),
            pltpu.SMEM((1,), jnp.float32),
        ],
    )(x)


def _fin_body(x_ref, p_ref, bdec_ref, tv_ref, sae_ref, fvu_ref, s_ref):
    i = pl.program_id(0)

    @pl.when(i == 0)
    def _():
        s_ref[0] = 0.0

    x = x_ref[...]
    sae = p_ref[...] + bdec_ref[...]
    sae_ref[...] = sae
    e = x - sae
    s_ref[0] += jnp.sum(e * e)
    nb = pl.num_programs(0)

    @pl.when(i == nb - 1)
    def _():
        fvu_ref[...] = jnp.full((1, 1), s_ref[0], jnp.float32) / tv_ref[...]


def _finalize(x, partial, b_dec2, tv):
    RB = 256
    return pl.pallas_call(
        _fin_body,
        grid=(B // RB,),
        in_specs=[
            pl.BlockSpec((RB, D), lambda i: (i, 0)),
            pl.BlockSpec((RB, D), lambda i: (i, 0)),
            pl.BlockSpec((1, D), lambda i: (0, 0)),
            pl.BlockSpec((1, 1), lambda i: (0, 0)),
        ],
        out_specs=[
            pl.BlockSpec((RB, D), lambda i: (i, 0)),
            pl.BlockSpec((1, 1), lambda i: (0, 0)),
        ],
        out_shape=[
            jax.ShapeDtypeStruct((B, D), jnp.float32),
            jax.ShapeDtypeStruct((1, 1), jnp.float32),
        ],
        scratch_shapes=[
            pltpu.SMEM((1,), jnp.float32),
        ],
    )(x, partial, b_dec2, tv)


def kernel(x, W_enc, b_enc, W_dec, b_dec):
    b_dec2 = b_dec.reshape(1, D)
    b_enc2 = b_enc.reshape(1, N)
    pre = _encode(x, W_enc, b_dec2, b_enc2)
    tv = _stats(x)
    top_acts, cols3, partial = _sc_topk_decode(pre, W_dec)
    top_indices = cols3.reshape(B, K)
    sae_out, fvu2 = _finalize(x, partial, b_dec2, tv)
    fvu = fvu2[0, 0]
    zero = jnp.zeros((), x.dtype)
    return (sae_out, top_acts, top_indices, fvu, zero, zero)
